# baseline (speedup 1.0000x reference)
import functools

import jax
import jax.numpy as jnp
from jax import lax
from jax.experimental import pallas as pl
from jax.experimental.pallas import tpu as pltpu
from jax.experimental.pallas import tpu_sc as plsc

EMBED_DIM = 32
LANES = 128
NW = 32


def _sc_gather(x1g, x2g, T1, T2):
  gpw = x1g.shape[0] // NW
  n = x1g.shape[0] * LANES
  mesh = plsc.VectorSubcoreMesh(core_axis_name="c", subcore_axis_name="s")

  @functools.partial(
      pl.kernel,
      out_type=jax.ShapeDtypeStruct((n, LANES), jnp.float32),
      mesh=mesh,
      compiler_params=pltpu.CompilerParams(use_tc_tiling_on_sc=False),
      scratch_types=(
          pltpu.VMEM((gpw, LANES), jnp.int32),
          pltpu.VMEM((gpw, LANES), jnp.int32),
          pltpu.VMEM((LANES, EMBED_DIM), jnp.float32),
          pltpu.VMEM((LANES, EMBED_DIM), jnp.float32),
          pltpu.SemaphoreType.DMA,
          pltpu.SemaphoreType.DMA,
      ),
  )
  def gather_kernel(x1_hbm, x2_hbm, t1_hbm, t2_hbm, ecat_hbm,
                    idx1_v, idx2_v, rows1_v, rows2_v, sem1, sem2):
    wid = lax.axis_index("s") * 2 + lax.axis_index("c")
    base = wid * gpw
    pltpu.sync_copy(x1_hbm.at[pl.ds(base, gpw)], idx1_v)
    pltpu.sync_copy(x2_hbm.at[pl.ds(base, gpw)], idx2_v)

    def body(g, carry):
      cp1 = pltpu.async_copy(t1_hbm.at[idx1_v.at[g]], rows1_v, sem1)
      cp2 = pltpu.async_copy(t2_hbm.at[idx2_v.at[g]], rows2_v, sem2)
      cp1.wait()
      cp2.wait()
      tok0 = (base + g) * LANES
      pltpu.sync_copy(rows1_v, ecat_hbm.at[pl.ds(tok0, LANES), pl.ds(0, EMBED_DIM)])
      pltpu.sync_copy(rows2_v,
                      ecat_hbm.at[pl.ds(tok0, LANES), pl.ds(EMBED_DIM, EMBED_DIM)])
      return carry

    lax.fori_loop(0, gpw, body, 0)

  return gather_kernel(x1g, x2g, T1, T2)


def _tc_mlp(ecat, w0, b0, w1, b1, w2, b2, wpt, bp):
  n = ecat.shape[0]
  blk = 8192
  grid = n // blk
  d = w1.shape[0]

  def mlp_kernel(ecat_ref, w0_ref, b0_ref, w1_ref, b1_ref,
                 w2_ref, b2_ref, wpt_ref, bp_ref, out_ref):
    e = ecat_ref[:, :d]
    x = jnp.dot(e, w0_ref[...], preferred_element_type=jnp.float32)
    x = jnp.maximum(x + b0_ref[...], 0.0)
    x = jnp.maximum(
        jnp.dot(x, w1_ref[...], preferred_element_type=jnp.float32) + b1_ref[...], 0.0)
    x = jnp.maximum(
        jnp.dot(x, w2_ref[...], preferred_element_type=jnp.float32) + b2_ref[...], 0.0)
    z = jnp.sum(x * wpt_ref[...], axis=1) + bp_ref[0, 0]
    out_ref[...] = jax.nn.sigmoid(z)

  full = lambda shape: pl.BlockSpec(shape, lambda i: (0,) * len(shape))
  return pl.pallas_call(
      mlp_kernel,
      grid=(grid,),
      in_specs=[
          pl.BlockSpec((blk, LANES), lambda i: (i, 0)),
          full((d, d)),
          full((1, d)),
          full((d, d)),
          full((1, d)),
          full((d, d)),
          full((1, d)),
          full((1, d)),
          full((1, 1)),
      ],
      out_specs=pl.BlockSpec((blk,), lambda i: (i,)),
      out_shape=jax.ShapeDtypeStruct((n,), jnp.float32),
  )(ecat, w0, b0, w1, b1, w2, b2, wpt, bp)


def kernel(x1, x2, T1, T2, W0, b0, W1, b1, W2, b2, Wp, bp):
  B, L = x1.shape
  n = B * L
  x1g = x1.astype(jnp.int32).reshape(n // LANES, LANES)
  x2g = x2.astype(jnp.int32).reshape(n // LANES, LANES)
  ecat = _sc_gather(x1g, x2g, T1, T2)
  out = _tc_mlp(
      ecat,
      W0, b0.reshape(1, -1),
      W1, b1.reshape(1, -1), W2, b2.reshape(1, -1),
      Wp.reshape(1, -1), bp.reshape(1, 1))
  return out.reshape(B, L)


# speedup vs baseline: 1.0253x; 1.0253x over previous
import functools

import jax
import jax.numpy as jnp
from jax import lax
from jax.experimental import pallas as pl
from jax.experimental.pallas import tpu as pltpu
from jax.experimental.pallas import tpu_sc as plsc

EMBED_DIM = 32
LANES = 128
NW = 32


def _sc_gather(x1i, x2i, T1, T2):
  bsz, seq = x1i.shape
  n = bsz * seq
  rpw = bsz // NW
  K = 8
  S = rpw // K
  mesh = plsc.VectorSubcoreMesh(core_axis_name="c", subcore_axis_name="s")

  @functools.partial(
      pl.kernel,
      out_type=jax.ShapeDtypeStruct((n, LANES), jnp.float32),
      mesh=mesh,
      compiler_params=pltpu.CompilerParams(use_tc_tiling_on_sc=False),
      scratch_types=(
          pltpu.VMEM((rpw, seq), jnp.int32),
          pltpu.VMEM((rpw, seq), jnp.int32),
          pltpu.VMEM((2, K, seq, EMBED_DIM), jnp.float32),
          pltpu.VMEM((2, K, seq, EMBED_DIM), jnp.float32),
          pltpu.SemaphoreType.DMA,
          pltpu.SemaphoreType.DMA,
          pltpu.SemaphoreType.DMA,
          pltpu.SemaphoreType.DMA,
      ),
  )
  def gather_kernel(x1_hbm, x2_hbm, t1_hbm, t2_hbm, ecat_hbm,
                    idx1_v, idx2_v, rows1_v, rows2_v, gs1, gs2, ws1, ws2):
    wid = lax.axis_index("s") * 2 + lax.axis_index("c")
    row0 = wid * rpw
    pltpu.sync_copy(x1_hbm.at[pl.ds(row0, rpw)], idx1_v)
    pltpu.sync_copy(x2_hbm.at[pl.ds(row0, rpw)], idx2_v)

    def out_slabs(s, j):
      tok0 = (row0 + s * K + j) * seq
      return (ecat_hbm.at[pl.ds(tok0, seq), pl.ds(0, EMBED_DIM)],
              ecat_hbm.at[pl.ds(tok0, seq), pl.ds(EMBED_DIM, EMBED_DIM)])

    def body(s, carry):
      b = lax.rem(s, 2)

      @pl.when(s >= 2)
      def _drain_writes():
        for j in range(K):
          o1, o2 = out_slabs(s - 2, j)
          pltpu.make_async_copy(rows1_v.at[b, j], o1, ws1).wait()
          pltpu.make_async_copy(rows2_v.at[b, j], o2, ws2).wait()

      cps = []
      for j in range(K):
        r = s * K + j
        cps.append(pltpu.async_copy(t1_hbm.at[idx1_v.at[r]], rows1_v.at[b, j], gs1))
        cps.append(pltpu.async_copy(t2_hbm.at[idx2_v.at[r]], rows2_v.at[b, j], gs2))
      for cp in cps:
        cp.wait()
      for j in range(K):
        o1, o2 = out_slabs(s, j)
        pltpu.async_copy(rows1_v.at[b, j], o1, ws1)
        pltpu.async_copy(rows2_v.at[b, j], o2, ws2)
      return carry

    lax.fori_loop(0, S, body, 0)

    for s_tail in (S - 2, S - 1):
      b = s_tail % 2
      for j in range(K):
        o1, o2 = out_slabs(s_tail, j)
        pltpu.make_async_copy(rows1_v.at[b, j], o1, ws1).wait()
        pltpu.make_async_copy(rows2_v.at[b, j], o2, ws2).wait()

  return gather_kernel(x1i, x2i, T1, T2)


def _tc_mlp(ecat, w0, b0, w1, b1, w2, b2, wpt, bp):
  n = ecat.shape[0]
  blk = 8192
  grid = n // blk
  d = w1.shape[0]

  def mlp_kernel(ecat_ref, w0_ref, b0_ref, w1_ref, b1_ref,
                 w2_ref, b2_ref, wpt_ref, bp_ref, out_ref):
    e = ecat_ref[:, :d]
    x = jnp.dot(e, w0_ref[...], preferred_element_type=jnp.float32)
    x = jnp.maximum(x + b0_ref[...], 0.0)
    x = jnp.maximum(
        jnp.dot(x, w1_ref[...], preferred_element_type=jnp.float32) + b1_ref[...], 0.0)
    x = jnp.maximum(
        jnp.dot(x, w2_ref[...], preferred_element_type=jnp.float32) + b2_ref[...], 0.0)
    z = jnp.sum(x * wpt_ref[...], axis=1) + bp_ref[0, 0]
    out_ref[...] = jax.nn.sigmoid(z)

  full = lambda shape: pl.BlockSpec(shape, lambda i: (0,) * len(shape))
  return pl.pallas_call(
      mlp_kernel,
      grid=(grid,),
      in_specs=[
          pl.BlockSpec((blk, LANES), lambda i: (i, 0)),
          full((d, d)),
          full((1, d)),
          full((d, d)),
          full((1, d)),
          full((d, d)),
          full((1, d)),
          full((1, d)),
          full((1, 1)),
      ],
      out_specs=pl.BlockSpec((blk,), lambda i: (i,)),
      out_shape=jax.ShapeDtypeStruct((n,), jnp.float32),
  )(ecat, w0, b0, w1, b1, w2, b2, wpt, bp)


def kernel(x1, x2, T1, T2, W0, b0, W1, b1, W2, b2, Wp, bp):
  B, L = x1.shape
  n = B * L
  ecat = _sc_gather(x1.astype(jnp.int32), x2.astype(jnp.int32), T1, T2)
  out = _tc_mlp(
      ecat,
      W0, b0.reshape(1, -1),
      W1, b1.reshape(1, -1), W2, b2.reshape(1, -1),
      Wp.reshape(1, -1), bp.reshape(1, 1))
  return out.reshape(B, L)
